# baseline (speedup 1.0000x reference)
import jax
import jax.numpy as jnp
from jax import lax
from jax.experimental import pallas as pl
from jax.experimental.pallas import tpu as pltpu
from jax.experimental.pallas import tpu_sc as plsc

MODEL_DIM = 768
HIDDEN_DIM = 3072
NUM_EXPERTS = 8
TOP_K = 2
SEQ = 2048

NC, NS, L = 2, 16, 16
NW = NC * NS
CHUNK = SEQ // NW
NCELL = NUM_EXPERTS * TOP_K
ROWBLK = 256
NH = 8
HBLK = HIDDEN_DIM // NH


def _gate_kernel(x_ref, gw_ref, gb_ref, lt_ref):
    logits = jax.lax.dot_general(
        x_ref[...], gw_ref[...], (((1,), (1,)), ((), ())),
        preferred_element_type=jnp.float32) + gb_ref[...]
    lt_ref[...] = jnp.concatenate(
        [logits, jnp.full((ROWBLK, L - NUM_EXPERTS), -jnp.inf, jnp.float32)],
        axis=1)


def _routing_kernel(lt_hbm, p_hbm, chunk_v, vec_v, shared_v):
    wid = lax.axis_index("s") * NC + lax.axis_index("c")
    base = wid * CHUNK
    pltpu.sync_copy(lt_hbm.at[pl.ds(base, CHUNK)], chunk_v)

    iota = lax.iota(jnp.int32, L)
    ninf = jnp.full((L,), -jnp.inf, jnp.float32)

    def tok(t, bv):
        v = chunk_v[t]
        m1 = v[0]
        i1 = jnp.int32(0)
        m2 = -jnp.inf
        i2 = jnp.int32(0)
        for e in range(1, NUM_EXPERTS):
            le = v[e]
            b1 = le > m1
            b2 = le > m2
            m2n = jnp.where(b1, m1, jnp.where(b2, le, m2))
            i2n = jnp.where(b1, i1, jnp.where(b2, e, i2))
            m1 = jnp.where(b1, le, m1)
            i1 = jnp.where(b1, e, i1)
            m2, i2 = m2n, i2n
        bv = jnp.where(iota == 2 * i1, jnp.full((L,), m1, jnp.float32), bv)
        bv = jnp.where(iota == 2 * i2 + 1, jnp.full((L,), m2, jnp.float32), bv)
        return bv

    bv = jax.lax.fori_loop(0, CHUNK, tok, ninf)

    vec_v[...] = bv
    pltpu.sync_copy(vec_v, shared_v.at[wid])
    plsc.subcore_barrier()

    @pl.when(wid == 0)
    def _():
        def merge(r, acc):
            pltpu.sync_copy(shared_v.at[r], vec_v)
            rv = vec_v[...]
            return jnp.where(rv > -jnp.inf, rv, acc)

        acc = jax.lax.fori_loop(0, NW, merge, ninf)
        vec_v[...] = acc
        pltpu.sync_copy(vec_v, p_hbm)


def _expert_kernel(x8_ref, fc1_w_ref, fc1_b_ref, fc2_w_ref, fc2_b_ref,
                   y_ref, nan_ref, h_s, y_s):
    g = pl.program_id(0)
    nan_ref[...] = jnp.full((ROWBLK, MODEL_DIM), jnp.nan, dtype=jnp.float32)

    @pl.when(g < NH)
    def _():
        for e in range(TOP_K):
            h = jax.lax.dot_general(
                x8_ref[...], fc1_w_ref[e], (((1,), (1,)), ((), ())),
                preferred_element_type=jnp.float32) + fc1_b_ref[e]
            h_s[e, :, g, :] = h * jax.nn.sigmoid(h)

    @pl.when(g >= 1)
    def _():
        for e in range(TOP_K):
            part = jax.lax.dot_general(
                h_s[e, :, g - 1, :], fc2_w_ref[e], (((1,), (1,)), ((), ())),
                preferred_element_type=jnp.float32)
            prev = jnp.where(g == 1, jnp.broadcast_to(
                fc2_b_ref[e], (NUM_EXPERTS, MODEL_DIM)), y_s[e])
            y_s[e] = prev + part

    @pl.when(g == NH)
    def _():
        y_ref[...] = y_s[...]


def _combine_kernel(cell_ref, y_ref, nan_in_ref, out_ref):
    v0 = cell_ref[:, 0:1]
    v1 = cell_ref[:, 1:2]
    m = jnp.maximum(v0, v1)
    e0 = jnp.exp(v0 - m)
    e1 = jnp.exp(v1 - m)
    denom = e0 + e1
    out_ref[...] = (e0 / denom) * y_ref[0] + (e1 / denom) * y_ref[1]


@jax.jit
def kernel(x, fc1_w, fc1_b, fc2_w, fc2_b, gate_w, gate_b):
    B, S, D = x.shape
    x2 = x.reshape(S, D)

    logits_pad = pl.pallas_call(
        _gate_kernel,
        grid=(S // ROWBLK,),
        in_specs=[
            pl.BlockSpec((ROWBLK, D), lambda i: (i, 0)),
            pl.BlockSpec((NUM_EXPERTS, D), lambda i: (0, 0)),
            pl.BlockSpec((1, NUM_EXPERTS), lambda i: (0, 0)),
        ],
        out_specs=pl.BlockSpec((ROWBLK, L), lambda i: (i, 0)),
        out_shape=jax.ShapeDtypeStruct((S, L), jnp.float32),
    )(x2, gate_w, gate_b.reshape(1, NUM_EXPERTS))

    y01, out_nan = pl.pallas_call(
        _expert_kernel,
        grid=(NH + 1,),
        in_specs=[
            pl.BlockSpec((NUM_EXPERTS, D), lambda g: (0, 0)),
            pl.BlockSpec((TOP_K, HBLK, D),
                         lambda g: (0, jnp.minimum(g, NH - 1), 0)),
            pl.BlockSpec((TOP_K, 1, HBLK),
                         lambda g: (0, 0, jnp.minimum(g, NH - 1))),
            pl.BlockSpec((TOP_K, D, HBLK),
                         lambda g: (0, 0, jnp.maximum(g - 1, 0))),
            pl.BlockSpec((TOP_K, 1, D), lambda g: (0, 0, 0)),
        ],
        out_specs=(
            pl.BlockSpec((TOP_K, NUM_EXPERTS, D), lambda g: (0, 0, 0)),
            pl.BlockSpec((ROWBLK, D), lambda g: (jnp.maximum(g - 1, 0), 0)),
        ),
        out_shape=(
            jax.ShapeDtypeStruct((TOP_K, NUM_EXPERTS, D), jnp.float32),
            jax.ShapeDtypeStruct((S, D), jnp.float32),
        ),
        scratch_shapes=[
            pltpu.VMEM((TOP_K, NUM_EXPERTS, NH, HBLK), jnp.float32),
            pltpu.VMEM((TOP_K, NUM_EXPERTS, D), jnp.float32),
        ],
    )(
        x2[0:NUM_EXPERTS, :],
        fc1_w,
        fc1_b.reshape(NUM_EXPERTS, 1, HIDDEN_DIM),
        fc2_w,
        fc2_b.reshape(NUM_EXPERTS, 1, MODEL_DIM),
    )

    mesh = plsc.VectorSubcoreMesh(core_axis_name="c", subcore_axis_name="s")
    cells = pl.kernel(
        _routing_kernel,
        mesh=mesh,
        out_type=jax.ShapeDtypeStruct((NCELL,), jnp.float32),
        scratch_types=[
            pltpu.VMEM((CHUNK, L), jnp.float32),
            pltpu.VMEM((NCELL,), jnp.float32),
            pltpu.VMEM_SHARED((NW, NCELL), jnp.float32),
        ],
    )(logits_pad)

    out = pl.pallas_call(
        _combine_kernel,
        grid=(1,),
        in_specs=[
            pl.BlockSpec((NUM_EXPERTS, TOP_K), lambda i: (0, 0)),
            pl.BlockSpec((TOP_K, NUM_EXPERTS, D), lambda i: (0, 0, 0)),
            pl.BlockSpec((NUM_EXPERTS, D), lambda i: (0, 0)),
        ],
        out_specs=pl.BlockSpec((NUM_EXPERTS, D), lambda i: (0, 0)),
        out_shape=jax.ShapeDtypeStruct((S, D), jnp.float32),
        input_output_aliases={2: 0},
    )(cells.reshape(NUM_EXPERTS, TOP_K), y01, out_nan)

    return out.reshape(B, S, D)


# speedup vs baseline: 1.1592x; 1.1592x over previous
import jax
import jax.numpy as jnp
from jax import lax
from jax.experimental import pallas as pl
from jax.experimental.pallas import tpu as pltpu
from jax.experimental.pallas import tpu_sc as plsc

MODEL_DIM = 768
HIDDEN_DIM = 3072
NUM_EXPERTS = 8
TOP_K = 2
SEQ = 2048

NC, NS, L = 2, 16, 16
NW = NC * NS
CHUNK = SEQ // NW
NCELL = NUM_EXPERTS * TOP_K
ROWBLK = 256
NH = 8
HBLK = HIDDEN_DIM // NH


def _gate_kernel(x_ref, gw_ref, gb_ref, lt_ref):
    logits = jax.lax.dot_general(
        x_ref[...], gw_ref[...], (((1,), (1,)), ((), ())),
        preferred_element_type=jnp.float32) + gb_ref[...]
    lt_ref[...] = jnp.concatenate(
        [logits, jnp.full((SEQ, L - NUM_EXPERTS), -jnp.inf, jnp.float32)],
        axis=1)


def _routing_kernel(lt_hbm, p_hbm, chunk_v, vec_v, shared_v):
    wid = lax.axis_index("s") * NC + lax.axis_index("c")
    base = wid * CHUNK
    pltpu.sync_copy(lt_hbm.at[pl.ds(base, CHUNK)], chunk_v)

    iota = lax.iota(jnp.int32, L)
    ninf = jnp.full((L,), -jnp.inf, jnp.float32)

    def tok(t, bv):
        v = chunk_v[t]
        m1 = v[0]
        i1 = jnp.int32(0)
        m2 = -jnp.inf
        i2 = jnp.int32(0)
        for e in range(1, NUM_EXPERTS):
            le = v[e]
            b1 = le > m1
            b2 = le > m2
            m2n = jnp.where(b1, m1, jnp.where(b2, le, m2))
            i2n = jnp.where(b1, i1, jnp.where(b2, e, i2))
            m1 = jnp.where(b1, le, m1)
            i1 = jnp.where(b1, e, i1)
            m2, i2 = m2n, i2n
        bv = jnp.where(iota == i1, jnp.full((L,), m1, jnp.float32), bv)
        bv = jnp.where(iota == NUM_EXPERTS + i2,
                       jnp.full((L,), m2, jnp.float32), bv)
        return bv

    bv = jax.lax.fori_loop(0, CHUNK, tok, ninf)

    vec_v[...] = bv
    pltpu.sync_copy(vec_v, shared_v.at[wid])
    plsc.subcore_barrier()

    @pl.when(wid == 0)
    def _():
        def merge(r, acc):
            pltpu.sync_copy(shared_v.at[r], vec_v)
            rv = vec_v[...]
            return jnp.where(rv > -jnp.inf, rv, acc)

        acc = jax.lax.fori_loop(0, NW, merge, ninf)
        vec_v[...] = acc
        pltpu.sync_copy(vec_v, p_hbm)


def _expert_kernel(x8_ref, fc1_w_ref, fc1_b_ref, fc2_w_ref, fc2_b_ref,
                   y_ref, nan_ref, h_s, y_s):
    g = pl.program_id(0)
    nan_ref[...] = jnp.full((ROWBLK, MODEL_DIM), jnp.nan, dtype=jnp.float32)

    @pl.when(g < NH)
    def _():
        for e in range(TOP_K):
            h = jax.lax.dot_general(
                x8_ref[...], fc1_w_ref[e], (((1,), (1,)), ((), ())),
                preferred_element_type=jnp.float32) + fc1_b_ref[e:e + 1, :]
            h_s[e, :, g, :] = h * jax.nn.sigmoid(h)

    @pl.when(g >= 1)
    def _():
        for e in range(TOP_K):
            part = jax.lax.dot_general(
                h_s[e, :, g - 1, :], fc2_w_ref[e], (((1,), (1,)), ((), ())),
                preferred_element_type=jnp.float32)
            prev = jnp.where(g == 1, jnp.broadcast_to(
                fc2_b_ref[e:e + 1, :], (NUM_EXPERTS, MODEL_DIM)), y_s[e])
            y_s[e] = prev + part

    @pl.when(g == NH)
    def _():
        y_ref[...] = y_s[...]


def _combine_kernel(cell_ref, y_ref, nan_in_ref, out_ref):
    E = NUM_EXPERTS
    v0 = cell_ref[0:1, 0:E]
    v1 = cell_ref[0:1, E:2 * E]
    m = jnp.maximum(v0, v1)
    e0 = jnp.exp(v0 - m)
    e1 = jnp.exp(v1 - m)
    denom = e0 + e1
    p0 = e0 / denom
    p1 = e1 / denom

    eye = (jax.lax.broadcasted_iota(jnp.int32, (E, E), 0) ==
           jax.lax.broadcasted_iota(jnp.int32, (E, E), 1))
    d0 = jnp.where(eye, jnp.broadcast_to(p0, (E, E)), 0.0)
    d1 = jnp.where(eye, jnp.broadcast_to(p1, (E, E)), 0.0)
    out_ref[...] = (
        jax.lax.dot_general(d0, y_ref[0], (((1,), (0,)), ((), ())),
                            preferred_element_type=jnp.float32) +
        jax.lax.dot_general(d1, y_ref[1], (((1,), (0,)), ((), ())),
                            preferred_element_type=jnp.float32))


@jax.jit
def kernel(x, fc1_w, fc1_b, fc2_w, fc2_b, gate_w, gate_b):
    B, S, D = x.shape
    x2 = x.reshape(S, D)

    logits_pad = pl.pallas_call(
        _gate_kernel,
        out_shape=jax.ShapeDtypeStruct((S, L), jnp.float32),
    )(x2, gate_w, gate_b.reshape(1, NUM_EXPERTS))

    y01, out_nan = pl.pallas_call(
        _expert_kernel,
        grid=(NH + 1,),
        in_specs=[
            pl.BlockSpec((NUM_EXPERTS, D), lambda g: (0, 0)),
            pl.BlockSpec((TOP_K, HBLK, D),
                         lambda g: (0, jnp.minimum(g, NH - 1), 0)),
            pl.BlockSpec((NUM_EXPERTS, HBLK),
                         lambda g: (0, jnp.minimum(g, NH - 1))),
            pl.BlockSpec((TOP_K, D, HBLK),
                         lambda g: (0, 0, jnp.maximum(g - 1, 0))),
            pl.BlockSpec((NUM_EXPERTS, D), lambda g: (0, 0)),
        ],
        out_specs=(
            pl.BlockSpec((TOP_K, NUM_EXPERTS, D), lambda g: (0, 0, 0)),
            pl.BlockSpec((ROWBLK, D), lambda g: (jnp.maximum(g - 1, 0), 0)),
        ),
        out_shape=(
            jax.ShapeDtypeStruct((TOP_K, NUM_EXPERTS, D), jnp.float32),
            jax.ShapeDtypeStruct((S, D), jnp.float32),
        ),
        scratch_shapes=[
            pltpu.VMEM((TOP_K, NUM_EXPERTS, NH, HBLK), jnp.float32),
            pltpu.VMEM((TOP_K, NUM_EXPERTS, D), jnp.float32),
        ],
    )(x2, fc1_w, fc1_b, fc2_w, fc2_b)

    mesh = plsc.VectorSubcoreMesh(core_axis_name="c", subcore_axis_name="s")
    cells = pl.kernel(
        _routing_kernel,
        mesh=mesh,
        out_type=jax.ShapeDtypeStruct((NCELL,), jnp.float32),
        scratch_types=[
            pltpu.VMEM((CHUNK, L), jnp.float32),
            pltpu.VMEM((NCELL,), jnp.float32),
            pltpu.VMEM_SHARED((NW, NCELL), jnp.float32),
        ],
    )(logits_pad)

    out = pl.pallas_call(
        _combine_kernel,
        grid=(1,),
        in_specs=[
            pl.BlockSpec((1, NCELL), lambda i: (0, 0)),
            pl.BlockSpec((TOP_K, NUM_EXPERTS, D), lambda i: (0, 0, 0)),
            pl.BlockSpec((NUM_EXPERTS, D), lambda i: (0, 0)),
        ],
        out_specs=pl.BlockSpec((NUM_EXPERTS, D), lambda i: (0, 0)),
        out_shape=jax.ShapeDtypeStruct((S, D), jnp.float32),
        input_output_aliases={2: 0},
    )(cells.reshape(1, NCELL), y01, out_nan)

    return out.reshape(B, S, D)


# speedup vs baseline: 1.1662x; 1.0061x over previous
import jax
import jax.numpy as jnp
from jax import lax
from jax.experimental import pallas as pl
from jax.experimental.pallas import tpu as pltpu
from jax.experimental.pallas import tpu_sc as plsc

MODEL_DIM = 768
HIDDEN_DIM = 3072
NUM_EXPERTS = 8
TOP_K = 2
SEQ = 2048

NC, NS, L = 2, 16, 16
NW = NC * NS
CHUNK = SEQ // NW
NCELL = NUM_EXPERTS * TOP_K
LPAD = 128
ROWBLK = 256
NH = 8
HBLK = HIDDEN_DIM // NH


def _gate_kernel(x_ref, gw_ref, gb_ref, lt_ref):
    logits = jax.lax.dot_general(
        x_ref[...].astype(jnp.bfloat16), gw_ref[...].astype(jnp.bfloat16),
        (((1,), (1,)), ((), ())),
        preferred_element_type=jnp.float32) + gb_ref[...]
    lt_ref[...] = jnp.concatenate(
        [logits, jnp.full((SEQ, LPAD - NUM_EXPERTS), -jnp.inf, jnp.float32)],
        axis=1)


def _routing_kernel(lt_hbm, p_hbm, chunk_v, vec_v):
    cid = lax.axis_index("c")
    sid = lax.axis_index("s")
    wid = cid * NS + sid
    base = wid * CHUNK
    pltpu.sync_copy(lt_hbm.at[pl.ds(base, CHUNK)], chunk_v)

    iota = lax.iota(jnp.int32, L)
    ninf = jnp.full((L,), -jnp.inf, jnp.float32)

    def tok(t, bv):
        v = chunk_v[t, pl.ds(0, L)]
        m1 = v[0]
        i1 = jnp.int32(0)
        m2 = -jnp.inf
        i2 = jnp.int32(0)
        for e in range(1, NUM_EXPERTS):
            le = v[e]
            b1 = le > m1
            b2 = le > m2
            m2n = jnp.where(b1, m1, jnp.where(b2, le, m2))
            i2n = jnp.where(b1, i1, jnp.where(b2, e, i2))
            m1 = jnp.where(b1, le, m1)
            i1 = jnp.where(b1, e, i1)
            m2, i2 = m2n, i2n
        bv = jnp.where(iota == i1, jnp.full((L,), m1, jnp.float32), bv)
        bv = jnp.where(iota == NUM_EXPERTS + i2,
                       jnp.full((L,), m2, jnp.float32), bv)
        return bv

    bv = jax.lax.fori_loop(0, CHUNK, tok, ninf)

    vec_v[...] = bv
    pltpu.sync_copy(vec_v, p_hbm.at[wid])


def _expert_kernel(x8_ref, fc1_w_ref, fc1_b_ref, fc2_w_ref, fc2_b_ref,
                   y_ref, nan_ref, h_s, y_s):
    g = pl.program_id(0)
    nan_ref[...] = jnp.full((ROWBLK, MODEL_DIM), jnp.nan, dtype=jnp.float32)

    @pl.when(g < NH)
    def _():
        for e in range(TOP_K):
            h = jax.lax.dot_general(
                x8_ref[...].astype(jnp.bfloat16),
                fc1_w_ref[e].astype(jnp.bfloat16), (((1,), (1,)), ((), ())),
                preferred_element_type=jnp.float32) + fc1_b_ref[e:e + 1, :]
            h_s[e, :, g, :] = h * jax.nn.sigmoid(h)

    @pl.when(g >= 1)
    def _():
        for e in range(TOP_K):
            part = jax.lax.dot_general(
                h_s[e, :, g - 1, :].astype(jnp.bfloat16),
                fc2_w_ref[e].astype(jnp.bfloat16), (((1,), (1,)), ((), ())),
                preferred_element_type=jnp.float32)
            prev = jnp.where(g == 1, jnp.broadcast_to(
                fc2_b_ref[e:e + 1, :], (NUM_EXPERTS, MODEL_DIM)), y_s[e])
            y_s[e] = prev + part

    @pl.when(g == NH)
    def _():
        y_ref[...] = y_s[...]


def _combine_kernel(cell_ref, y_ref, nan_in_ref, out_ref):
    E = NUM_EXPERTS
    cells = cell_ref[0:1, :]
    for r in range(1, NW):
        rv = cell_ref[r:r + 1, :]
        cells = jnp.where(rv > -jnp.inf, rv, cells)
    v0 = cells[0:1, 0:E]
    v1 = cells[0:1, E:2 * E]
    m = jnp.maximum(v0, v1)
    e0 = jnp.exp(v0 - m)
    e1 = jnp.exp(v1 - m)
    denom = e0 + e1
    p0 = e0 / denom
    p1 = e1 / denom

    eye = (jax.lax.broadcasted_iota(jnp.int32, (E, E), 0) ==
           jax.lax.broadcasted_iota(jnp.int32, (E, E), 1))
    d0 = jnp.where(eye, jnp.broadcast_to(p0, (E, E)), 0.0)
    d1 = jnp.where(eye, jnp.broadcast_to(p1, (E, E)), 0.0)
    out_ref[...] = (
        jax.lax.dot_general(d0, y_ref[0], (((1,), (0,)), ((), ())),
                            preferred_element_type=jnp.float32) +
        jax.lax.dot_general(d1, y_ref[1], (((1,), (0,)), ((), ())),
                            preferred_element_type=jnp.float32))


@jax.jit
def kernel(x, fc1_w, fc1_b, fc2_w, fc2_b, gate_w, gate_b):
    B, S, D = x.shape
    x2 = x.reshape(S, D)

    logits_pad = pl.pallas_call(
        _gate_kernel,
        out_shape=jax.ShapeDtypeStruct((S, LPAD), jnp.float32),
    )(x2, gate_w, gate_b.reshape(1, NUM_EXPERTS))

    y01, out_nan = pl.pallas_call(
        _expert_kernel,
        grid=(NH + 1,),
        in_specs=[
            pl.BlockSpec((NUM_EXPERTS, D), lambda g: (0, 0)),
            pl.BlockSpec((TOP_K, HBLK, D),
                         lambda g: (0, jnp.minimum(g, NH - 1), 0)),
            pl.BlockSpec((NUM_EXPERTS, HBLK),
                         lambda g: (0, jnp.minimum(g, NH - 1))),
            pl.BlockSpec((TOP_K, D, HBLK),
                         lambda g: (0, 0, jnp.maximum(g - 1, 0))),
            pl.BlockSpec((NUM_EXPERTS, D), lambda g: (0, 0)),
        ],
        out_specs=(
            pl.BlockSpec((TOP_K, NUM_EXPERTS, D), lambda g: (0, 0, 0)),
            pl.BlockSpec((ROWBLK, D), lambda g: (jnp.maximum(g - 1, 0), 0)),
        ),
        out_shape=(
            jax.ShapeDtypeStruct((TOP_K, NUM_EXPERTS, D), jnp.float32),
            jax.ShapeDtypeStruct((S, D), jnp.float32),
        ),
        scratch_shapes=[
            pltpu.VMEM((TOP_K, NUM_EXPERTS, NH, HBLK), jnp.float32),
            pltpu.VMEM((TOP_K, NUM_EXPERTS, D), jnp.float32),
        ],
    )(x2, fc1_w, fc1_b, fc2_w, fc2_b)

    mesh = plsc.VectorSubcoreMesh(core_axis_name="c", subcore_axis_name="s")
    cells = pl.kernel(
        _routing_kernel,
        mesh=mesh,
        out_type=jax.ShapeDtypeStruct((NW, NCELL), jnp.float32),
        scratch_types=[
            pltpu.VMEM((CHUNK, LPAD), jnp.float32),
            pltpu.VMEM((NCELL,), jnp.float32),
        ],
    )(logits_pad)

    out = pl.pallas_call(
        _combine_kernel,
        grid=(1,),
        in_specs=[
            pl.BlockSpec((NW, NCELL), lambda i: (0, 0)),
            pl.BlockSpec((TOP_K, NUM_EXPERTS, D), lambda i: (0, 0, 0)),
            pl.BlockSpec((NUM_EXPERTS, D), lambda i: (0, 0)),
        ],
        out_specs=pl.BlockSpec((NUM_EXPERTS, D), lambda i: (0, 0)),
        out_shape=jax.ShapeDtypeStruct((S, D), jnp.float32),
        input_output_aliases={2: 0},
    )(cells, y01, out_nan)

    return out.reshape(B, S, D)
